# Initial kernel scaffold; baseline (speedup 1.0000x reference)
import jax
import jax.numpy as jnp
from jax.experimental import pallas as pl


def kernel(pooled_feat, W1, b1, W2, b2):
    raise NotImplementedError("write your pallas kernel here")


# speedup vs baseline: 2.4924x; 2.4924x over previous
import jax
import jax.numpy as jnp
from jax.experimental import pallas as pl
from jax.experimental.pallas import tpu as pltpu

_INV_SQRT2 = 0.7071067811865476


def _router_body(x_ref, w1_ref, b1_ref, w2_ref, b2_ref,
                 probs_ref, rw_ref, idx_ref):
    x = x_ref[...]
    h = jnp.dot(x, w1_ref[...], preferred_element_type=jnp.float32)
    h = h + b1_ref[...]
    h = 0.5 * h * (1.0 + jax.lax.erf(h * _INV_SQRT2))
    logits = jnp.dot(h, w2_ref[...], preferred_element_type=jnp.float32)
    logits = logits + b2_ref[...]

    m = jnp.max(logits, axis=-1, keepdims=True)
    e = jnp.exp(logits - m)
    s = jnp.sum(e, axis=-1, keepdims=True)
    probs = e / s
    probs_ref[...] = probs

    ne = probs.shape[-1]
    iota = jax.lax.broadcasted_iota(jnp.int32, probs.shape, 1)
    m1 = jnp.max(probs, axis=-1, keepdims=True)
    i1 = jnp.min(jnp.where(probs == m1, iota, ne), axis=-1, keepdims=True)
    masked = jnp.where(iota == i1, -1.0, probs)
    m2 = jnp.max(masked, axis=-1, keepdims=True)
    i2 = jnp.min(jnp.where(masked == m2, iota, ne), axis=-1, keepdims=True)
    denom = jnp.maximum(m1 + m2, 1e-6)
    rw_ref[...] = jnp.concatenate([m1 / denom, m2 / denom], axis=-1)
    idx_ref[...] = jnp.concatenate([i1, i2], axis=-1).astype(jnp.int32)


def kernel(pooled_feat, W1, b1, W2, b2):
    B, D = pooled_feat.shape
    H = W1.shape[1]
    NE = W2.shape[1]
    BS = 1024
    grid = (B // BS,)

    b1r = b1.reshape(1, H)
    b2r = b2.reshape(1, NE)

    probs, rw, idx = pl.pallas_call(
        _router_body,
        grid=grid,
        in_specs=[
            pl.BlockSpec((BS, D), lambda i: (i, 0)),
            pl.BlockSpec((D, H), lambda i: (0, 0)),
            pl.BlockSpec((1, H), lambda i: (0, 0)),
            pl.BlockSpec((H, NE), lambda i: (0, 0)),
            pl.BlockSpec((1, NE), lambda i: (0, 0)),
        ],
        out_specs=[
            pl.BlockSpec((BS, NE), lambda i: (i, 0)),
            pl.BlockSpec((BS, 2), lambda i: (i, 0)),
            pl.BlockSpec((BS, 2), lambda i: (i, 0)),
        ],
        out_shape=[
            jax.ShapeDtypeStruct((B, NE), jnp.float32),
            jax.ShapeDtypeStruct((B, 2), jnp.float32),
            jax.ShapeDtypeStruct((B, 2), jnp.int32),
        ],
        compiler_params=pltpu.CompilerParams(
            dimension_semantics=("parallel",),
        ),
    )(pooled_feat, W1, b1r, W2, b2r)

    return (rw, idx, probs)


# speedup vs baseline: 2.6097x; 1.0471x over previous
import jax
import jax.numpy as jnp
from jax.experimental import pallas as pl
from jax.experimental.pallas import tpu as pltpu

_INV_SQRT2 = 0.7071067811865476


def _router_body(x_ref, w1_ref, b1_ref, w2_ref, b2_ref,
                 probs_ref, rw_ref, idx_ref):
    x = x_ref[...]
    h = jnp.dot(x, w1_ref[...], preferred_element_type=jnp.float32)
    h = h + b1_ref[...]
    h = 0.5 * h * (1.0 + jax.lax.erf(h * _INV_SQRT2))
    logits = jnp.dot(h, w2_ref[...], preferred_element_type=jnp.float32)
    logits = logits + b2_ref[...]

    m = jnp.max(logits, axis=-1, keepdims=True)
    e = jnp.exp(logits - m)
    s = jnp.sum(e, axis=-1, keepdims=True)
    probs = e / s
    probs_ref[...] = probs

    ne = probs.shape[-1]
    iota = jax.lax.broadcasted_iota(jnp.int32, probs.shape, 1)
    m1 = jnp.max(probs, axis=-1, keepdims=True)
    i1 = jnp.min(jnp.where(probs == m1, iota, ne), axis=-1, keepdims=True)
    masked = jnp.where(iota == i1, -1.0, probs)
    m2 = jnp.max(masked, axis=-1, keepdims=True)
    i2 = jnp.min(jnp.where(masked == m2, iota, ne), axis=-1, keepdims=True)
    denom = jnp.maximum(m1 + m2, 1e-6)
    rw_ref[...] = jnp.concatenate([m1 / denom, m2 / denom], axis=-1)
    idx_ref[...] = jnp.concatenate([i1, i2], axis=-1).astype(jnp.int32)


def kernel(pooled_feat, W1, b1, W2, b2):
    B, D = pooled_feat.shape
    H = W1.shape[1]
    NE = W2.shape[1]
    BS = 2048
    grid = (B // BS,)

    b1r = b1.reshape(1, H)
    b2r = b2.reshape(1, NE)

    probs, rw, idx = pl.pallas_call(
        _router_body,
        grid=grid,
        in_specs=[
            pl.BlockSpec((BS, D), lambda i: (i, 0)),
            pl.BlockSpec((D, H), lambda i: (0, 0)),
            pl.BlockSpec((1, H), lambda i: (0, 0)),
            pl.BlockSpec((H, NE), lambda i: (0, 0)),
            pl.BlockSpec((1, NE), lambda i: (0, 0)),
        ],
        out_specs=[
            pl.BlockSpec((BS, NE), lambda i: (i, 0)),
            pl.BlockSpec((BS, 2), lambda i: (i, 0)),
            pl.BlockSpec((BS, 2), lambda i: (i, 0)),
        ],
        out_shape=[
            jax.ShapeDtypeStruct((B, NE), jnp.float32),
            jax.ShapeDtypeStruct((B, 2), jnp.float32),
            jax.ShapeDtypeStruct((B, 2), jnp.int32),
        ],
        compiler_params=pltpu.CompilerParams(
            dimension_semantics=("parallel",),
        ),
    )(pooled_feat, W1, b1r, W2, b2r)

    return (rw, idx, probs)
